# baseline (speedup 1.0000x reference)
import jax
import jax.numpy as jnp
from jax.experimental import pallas as pl

B, S, D = 4, 8192, 1024
BLK = 512


def _body(x_ref, m_ref, e_ref, o_ref):
    o_ref[...] = jnp.where(m_ref[...] == 0, 0.0, x_ref[...] + e_ref[...])


def kernel(inputs, input_mask, pos_emb):
    x = inputs.reshape(B * S, D)
    m = input_mask.reshape(B * S, 1)
    nchunk = S // BLK
    out = pl.pallas_call(
        _body,
        grid=(nchunk, B),
        in_specs=[
            pl.BlockSpec((BLK, D), lambda i, j: (j * nchunk + i, 0)),
            pl.BlockSpec((BLK, 1), lambda i, j: (j * nchunk + i, 0)),
            pl.BlockSpec((BLK, D), lambda i, j: (i, 0)),
        ],
        out_specs=pl.BlockSpec((BLK, D), lambda i, j: (j * nchunk + i, 0)),
        out_shape=jax.ShapeDtypeStruct((B * S, D), jnp.float32),
    )(x, m, pos_emb[:S])
    return out.reshape(B, S, D)


# speedup vs baseline: 1.1099x; 1.1099x over previous
import jax
import jax.numpy as jnp
from jax.experimental import pallas as pl

B, S, D = 4, 8192, 1024
BLK = 1024


def _body(x_ref, m_ref, e_ref, o_ref):
    o_ref[...] = jnp.where(m_ref[...] == 0, 0.0, x_ref[...] + e_ref[...])


def kernel(inputs, input_mask, pos_emb):
    x = inputs.reshape(B * S, D)
    m = input_mask.reshape(B * S, 1)
    nchunk = S // BLK
    out = pl.pallas_call(
        _body,
        grid=(nchunk, B),
        in_specs=[
            pl.BlockSpec((BLK, D), lambda i, j: (j * nchunk + i, 0)),
            pl.BlockSpec((BLK, 1), lambda i, j: (j * nchunk + i, 0)),
            pl.BlockSpec((BLK, D), lambda i, j: (i, 0)),
        ],
        out_specs=pl.BlockSpec((BLK, D), lambda i, j: (j * nchunk + i, 0)),
        out_shape=jax.ShapeDtypeStruct((B * S, D), jnp.float32),
    )(x, m, pos_emb[:S])
    return out.reshape(B, S, D)


# speedup vs baseline: 1.1403x; 1.0274x over previous
import jax
import jax.numpy as jnp
from jax.experimental import pallas as pl

B, S, D = 4, 8192, 1024
BLK = 2048


def _body(x_ref, m_ref, e_ref, o_ref):
    o_ref[...] = jnp.where(m_ref[...] == 0, 0.0, x_ref[...] + e_ref[...])


def kernel(inputs, input_mask, pos_emb):
    x = inputs.reshape(B * S, D)
    m = input_mask.reshape(B * S, 1)
    nchunk = S // BLK
    out = pl.pallas_call(
        _body,
        grid=(nchunk, B),
        in_specs=[
            pl.BlockSpec((BLK, D), lambda i, j: (j * nchunk + i, 0)),
            pl.BlockSpec((BLK, 1), lambda i, j: (j * nchunk + i, 0)),
            pl.BlockSpec((BLK, D), lambda i, j: (i, 0)),
        ],
        out_specs=pl.BlockSpec((BLK, D), lambda i, j: (j * nchunk + i, 0)),
        out_shape=jax.ShapeDtypeStruct((B * S, D), jnp.float32),
    )(x, m, pos_emb[:S])
    return out.reshape(B, S, D)
